# Initial kernel scaffold; baseline (speedup 1.0000x reference)
import jax
import jax.numpy as jnp
from jax.experimental import pallas as pl


def kernel(node_s, seq, edge_s, params, edge_index):
    raise NotImplementedError("write your pallas kernel here")


# speedup vs baseline: 3.0133x; 3.0133x over previous
import functools

import jax
import jax.numpy as jnp
import numpy as np
from jax import lax
from jax.experimental import pallas as pl
from jax.experimental.pallas import tpu as pltpu
from jax.experimental.pallas import tpu_sc as plsc

N_GRAPHS = 32
NODES_PER_GRAPH = 70
N = N_GRAPHS * NODES_PER_GRAPH
NP = 2304
E = 71680
GCN_OUT = [128, 256, 256]

NTILES = 32
EPT = E // NTILES
EB = 224
NBLK = EPT // EB
ROWS_PP = 40
F32 = jnp.float32

_mesh = plsc.VectorSubcoreMesh(core_axis_name="c", subcore_axis_name="s")


def _proj_body(x_ref, w_ref, b_ref, o_ref):
    o_ref[...] = (
        jnp.dot(x_ref[...], w_ref[...], preferred_element_type=F32) + b_ref[...]
    )


def _edge_proj(edge_s, w, b):
    blk = 2048
    return pl.pallas_call(
        _proj_body,
        grid=(E // blk,),
        in_specs=[
            pl.BlockSpec((blk, 39), lambda i: (i, 0)),
            pl.BlockSpec((39, 128), lambda i: (0, 0)),
            pl.BlockSpec((1, 128), lambda i: (0, 0)),
        ],
        out_specs=pl.BlockSpec((blk, 128), lambda i: (i, 0)),
        out_shape=jax.ShapeDtypeStruct((E, 128), F32),
    )(edge_s, w, b.reshape(1, 128))


def _node_proj(x0, w, b):
    return pl.pallas_call(
        _proj_body,
        out_shape=jax.ShapeDtypeStruct((NP, 128), F32),
    )(x0, w, b.reshape(1, 128))


def _prep_body(x_ref, wq_ref, bq_ref, wk_ref, bk_ref, wv_ref, bv_ref,
               wet_ref, a_ref, qws_ref, v_ref, *, sc):
    x = x_ref[...]
    q = jnp.dot(x, wq_ref[...], preferred_element_type=F32) + bq_ref[...]
    k = jnp.dot(x, wk_ref[...], preferred_element_type=F32) + bk_ref[...]
    v_ref[...] = jnp.dot(x, wv_ref[...], preferred_element_type=F32) + bv_ref[...]
    a_ref[...] = lax.dot_general(
        q, k, (((1,), (1,)), ((), ())), preferred_element_type=F32) * sc
    qws_ref[...] = jnp.dot(q, wet_ref[...], preferred_element_type=F32) * sc


def _prep(x, lp, fo):
    sc = 1.0 / np.sqrt(float(fo))
    return pl.pallas_call(
        functools.partial(_prep_body, sc=sc),
        out_shape=[
            jax.ShapeDtypeStruct((NP, NP), F32),
            jax.ShapeDtypeStruct((NP, 128), F32),
            jax.ShapeDtypeStruct((NP, fo), F32),
        ],
    )(x, lp['Wq'], lp['bq'].reshape(1, fo), lp['Wk'], lp['bk'].reshape(1, fo),
      lp['Wv'], lp['bv'].reshape(1, fo), lp['We'].T)


def _combine_body(s_ref, v_ref, acc_ref, den_ref, x_ref, we_ref, wsk_ref,
                  bsk_ref, o_ref, *, blk):
    i = pl.program_id(0)
    msg = jnp.dot(s_ref[...], v_ref[...], preferred_element_type=F32)
    acc = acc_ref[0] + acc_ref[1]
    msg = msg + jnp.dot(acc, we_ref[...], preferred_element_type=F32)
    den = den_ref[0] + den_ref[1]
    out = msg * (1.0 / (den + 1e-16))[:, None]
    out = out + jnp.dot(x_ref[...], wsk_ref[...], preferred_element_type=F32)
    out = out + bsk_ref[...]
    out = jnp.where(out >= 0.0, out, 0.01 * out)
    rows = i * blk + lax.broadcasted_iota(jnp.int32, (blk, 1), 0)
    o_ref[...] = jnp.where(rows < N, out, 0.0)


def _combine(S, v, acc, den, x, lp, fi, fo):
    blk = 256
    return pl.pallas_call(
        functools.partial(_combine_body, blk=blk),
        grid=(NP // blk,),
        in_specs=[
            pl.BlockSpec((blk, NP), lambda i: (i, 0)),
            pl.BlockSpec((NP, fo), lambda i: (0, 0)),
            pl.BlockSpec((2, blk, 128), lambda i: (0, i, 0)),
            pl.BlockSpec((2, blk), lambda i: (0, i)),
            pl.BlockSpec((blk, fi), lambda i: (i, 0)),
            pl.BlockSpec((128, fo), lambda i: (0, 0)),
            pl.BlockSpec((fi, fo), lambda i: (0, 0)),
            pl.BlockSpec((1, fo), lambda i: (0, 0)),
        ],
        out_specs=pl.BlockSpec((blk, fo), lambda i: (i, 0)),
        out_shape=jax.ShapeDtypeStruct((NP, fo), F32),
    )(S, v, acc, den, x, lp['We'], lp['Wskip'], lp['bskip'].reshape(1, fo))


def _head_body(x_ref, p_ref, w1_ref, b1_ref, w2_ref, b2_ref, w3_ref, b3_ref,
               o_ref):
    px = jnp.dot(p_ref[...], x_ref[...], preferred_element_type=F32)
    h = jnp.maximum(jnp.dot(px, w1_ref[...], preferred_element_type=F32)
                    + b1_ref[...], 0.0)
    h = jnp.maximum(jnp.dot(h, w2_ref[...], preferred_element_type=F32)
                    + b2_ref[...], 0.0)
    o = jnp.dot(h, w3_ref[...], preferred_element_type=F32) + b3_ref[...]
    o_ref[...] = jax.nn.sigmoid(o)


def _head(x, pool, params):
    return pl.pallas_call(
        _head_body,
        out_shape=jax.ShapeDtypeStruct((N_GRAPHS, 1), F32),
    )(x, pool, params['fc1_W'], params['fc1_b'].reshape(1, 128),
      params['fc2_W'], params['fc2_b'].reshape(1, 64),
      params['fc3_W'], params['fc3_b'].reshape(1, 1))


@functools.partial(
    pl.kernel, mesh=_mesh,
    compiler_params=pltpu.CompilerParams(needs_layout_passes=False),
    out_type=[
        jax.ShapeDtypeStruct((E,), F32),
        jax.ShapeDtypeStruct((2, NP), F32),
        jax.ShapeDtypeStruct((2, NP, 128), F32),
    ],
    scratch_types=[
        pltpu.VMEM((2, 112), jnp.int32),
        pltpu.VMEM((224,), jnp.int32),
        pltpu.VMEM((224,), jnp.int32),
        pltpu.VMEM((224,), jnp.int32),
        pltpu.VMEM((224,), F32),
        pltpu.VMEM((224, 128), F32),
        pltpu.VMEM((224, 128), F32),
        pltpu.VMEM((224,), F32),
        pltpu.VMEM((NP,), F32),
        pltpu.VMEM((16, 128), F32),
        pltpu.VMEM((128,), F32),
        pltpu.VMEM_SHARED((NP, 128), F32),
        pltpu.VMEM_SHARED((16, NP), F32),
        pltpu.SemaphoreType.DMA,
        pltpu.SemaphoreType.DMA,
    ],
)
def _edge_kernel(ea_hbm, afl_hbm, qws_hbm, src_hbm, dst_hbm,
                 exv_hbm, den_hbm, acc_hbm,
                 idx2, dstv, srcv, rowix, avr, qg, eav, exb,
                 den_l, denc, denout, accSp, denStg, sem, sem2):
    c = lax.axis_index("c")
    s = lax.axis_index("s")
    iota = lax.iota(jnp.int32, 16)
    z16 = jnp.zeros((16,), F32)
    zi16 = jnp.zeros((16,), jnp.int32)

    def zden(i, _):
        den_l[pl.ds(i * 16, 16)] = z16
        return 0
    lax.fori_loop(0, NP // 16, zden, 0)

    def zrow(i, _):
        for kk in range(8):
            eav[i, pl.ds(16 * kk, 16)] = z16
        return 0
    lax.fori_loop(0, 144, zrow, 0)
    pltpu.sync_copy(eav.at[pl.ds(0, 144)], accSp.at[pl.ds(s * 144, 144)])
    plsc.subcore_barrier()

    base = (c * 16 + s) * EPT

    def block(bb, _):
        be = base + bb * EB
        pltpu.sync_copy(dst_hbm.at[pl.ds(be, 112)], idx2.at[0])
        pltpu.sync_copy(dst_hbm.at[pl.ds(be + 112, 112)], idx2.at[1])
        pltpu.sync_copy(dst_hbm.at[pl.ds(be, 224)], dstv)
        pltpu.sync_copy(src_hbm.at[pl.ds(be, 224)], srcv)
        pltpu.sync_copy(ea_hbm.at[pl.ds(be, 224)], eav)

        def gidx(g, _):
            d16 = dstv[pl.ds(g * 16, 16)]
            s16 = srcv[pl.ds(g * 16, 16)]
            rowix[pl.ds(g * 16, 16)] = d16 * NP + s16
            return 0
        lax.fori_loop(0, 14, gidx, 0)

        cp1 = pltpu.async_copy(afl_hbm.at[rowix.at[pl.ds(0, 112)]],
                               avr.at[pl.ds(0, 112)], sem)
        cp2 = pltpu.async_copy(afl_hbm.at[rowix.at[pl.ds(112, 112)]],
                               avr.at[pl.ds(112, 112)], sem)
        cp3 = pltpu.async_copy(qws_hbm.at[idx2.at[0]], qg.at[pl.ds(0, 112)], sem2)
        cp4 = pltpu.async_copy(qws_hbm.at[idx2.at[1]], qg.at[pl.ds(112, 112)], sem2)
        cp1.wait(); cp2.wait(); cp3.wait(); cp4.wait()

        def group(g, _):
            e0 = g * 16
            rows = e0 + iota
            acc = avr[pl.ds(e0, 16)]

            def dim(d, acc):
                cols = zi16 + d
                a = plsc.load_gather(eav, [rows, cols])
                b = plsc.load_gather(qg, [rows, cols])
                return acc + a * b
            acc = lax.fori_loop(0, 128, dim, acc)
            exvec = jnp.exp(acc)
            exb[pl.ds(e0, 16)] = exvec
            plsc.addupdate_scatter(den_l, [dstv[pl.ds(e0, 16)]], exvec)
            return 0
        lax.fori_loop(0, 14, group, 0)

        def scale(e, _):
            exsplat = plsc.load_gather(exb, [zi16 + e])
            for kk in range(8):
                eav[e, pl.ds(16 * kk, 16)] = (
                    eav[e, pl.ds(16 * kk, 16)] * exsplat)
            return 0
        lax.fori_loop(0, EB, scale, 0)

        pltpu.sync_copy(eav.at[pl.ds(0, 112)], accSp.at[idx2.at[0]], add=True)
        pltpu.sync_copy(eav.at[pl.ds(112, 112)], accSp.at[idx2.at[1]], add=True)
        pltpu.sync_copy(exb, exv_hbm.at[pl.ds(be, 224)])
        return 0
    lax.fori_loop(0, NBLK, block, 0)

    pltpu.sync_copy(den_l, denStg.at[s])
    plsc.subcore_barrier()

    def _reduce_chunk(ch):
        pltpu.sync_copy(denStg.at[:, pl.ds(ch * 128, 128)], denc)

        def dred(m, _):
            acc = z16
            for r in range(16):
                acc = acc + denc[r, pl.ds(m * 16, 16)]
            denout[pl.ds(m * 16, 16)] = acc
            return 0
        lax.fori_loop(0, 8, dred, 0)
        pltpu.sync_copy(denout, den_hbm.at[c, pl.ds(ch * 128, 128)])

    _reduce_chunk(s)

    @pl.when(s < NP // 128 - 16)
    def _():
        _reduce_chunk(s + 16)

    pltpu.sync_copy(accSp.at[pl.ds(s * 144, 144)],
                    acc_hbm.at[c, pl.ds(s * 144, 144)])


@functools.partial(
    pl.kernel, mesh=_mesh,
    compiler_params=pltpu.CompilerParams(needs_layout_passes=False),
    out_type=jax.ShapeDtypeStruct((NP, NP), F32),
    scratch_types=[
        pltpu.VMEM((ROWS_PP, NP), F32),
        pltpu.VMEM((1024,), jnp.int32),
        pltpu.VMEM((1024,), jnp.int32),
        pltpu.VMEM((1024,), F32),
    ],
)
def _sbuild_kernel(src_hbm, dst_hbm, exv_hbm, zr_hbm, s_hbm,
                   sblk, srcb, dstb, exbb):
    c = lax.axis_index("c")
    s = lax.axis_index("s")
    wid = c * 16 + s

    for off, nrows in ((0, 40), (40, 32)):
        rowbase = wid * 72 + off
        pltpu.sync_copy(zr_hbm.at[pl.ds(0, nrows)], sblk.at[pl.ds(0, nrows)])

        def eblock(b, _, rowbase=rowbase, nrows=nrows):
            eb = b * 1024
            pltpu.sync_copy(dst_hbm.at[pl.ds(eb, 1024)], dstb)
            pltpu.sync_copy(src_hbm.at[pl.ds(eb, 1024)], srcb)
            pltpu.sync_copy(exv_hbm.at[pl.ds(eb, 1024)], exbb)

            def grp(g, _):
                d16 = dstb[pl.ds(g * 16, 16)]
                s16 = srcb[pl.ds(g * 16, 16)]
                x16 = exbb[pl.ds(g * 16, 16)]
                row = d16 - rowbase
                m = jnp.logical_and(d16 >= rowbase, d16 < rowbase + nrows)
                plsc.addupdate_scatter(sblk, [row, s16], x16, mask=m)
                return 0
            lax.fori_loop(0, 64, grp, 0)
            return 0
        lax.fori_loop(0, E // 1024, eblock, 0)
        pltpu.sync_copy(sblk.at[pl.ds(0, nrows)],
                        s_hbm.at[pl.ds(rowbase, nrows)])


def kernel(node_s, seq, edge_s, params, edge_index):
    src = edge_index[0]
    dst = edge_index[1]
    x0 = jnp.concatenate([node_s, seq], axis=-1).reshape(N, 26)
    x0 = jnp.pad(x0, ((0, NP - N), (0, 0)))
    x = _node_proj(x0, params['proj_node_W'], params['proj_node_b'])
    ea = _edge_proj(edge_s, params['proj_edge_W'], params['proj_edge_b'])
    zr = jnp.zeros((ROWS_PP, NP), F32)

    fis = [128, 128, 256]
    for li, (lp, fo) in enumerate(zip(params['gcn'], GCN_OUT)):
        fi = fis[li]
        A, qws, v = _prep(x, lp, fo)
        afl = A.reshape(NP * NP)
        exv, den, acc = _edge_kernel(ea, afl, qws, src, dst)
        del A
        S = _sbuild_kernel(src, dst, exv, zr)
        x = _combine(S, v, acc, den, x, lp, fi, fo)

    pool = jnp.kron(jnp.eye(N_GRAPHS, dtype=F32),
                    jnp.full((1, NODES_PER_GRAPH), 1.0 / NODES_PER_GRAPH, F32))
    return _head(x[:N], pool, params)


# speedup vs baseline: 4.4307x; 1.4704x over previous
import functools

import jax
import jax.numpy as jnp
import numpy as np
from jax import lax
from jax.experimental import pallas as pl
from jax.experimental.pallas import tpu as pltpu
from jax.experimental.pallas import tpu_sc as plsc

N_GRAPHS = 32
NODES_PER_GRAPH = 70
N = N_GRAPHS * NODES_PER_GRAPH
NP = 2304
E = 71680
GCN_OUT = [128, 256, 256]

NTILES = 32
EPT = E // NTILES
EB = 224
NBLK = EPT // EB
ROWS_PP = 40
F32 = jnp.float32

_mesh = plsc.VectorSubcoreMesh(core_axis_name="c", subcore_axis_name="s")


def _proj_body(x_ref, w_ref, b_ref, o_ref):
    o_ref[...] = (
        jnp.dot(x_ref[...], w_ref[...], preferred_element_type=F32) + b_ref[...]
    )


def _edge_proj(edge_s, w, b):
    blk = 2048
    return pl.pallas_call(
        _proj_body,
        grid=(E // blk,),
        in_specs=[
            pl.BlockSpec((blk, 39), lambda i: (i, 0)),
            pl.BlockSpec((39, 128), lambda i: (0, 0)),
            pl.BlockSpec((1, 128), lambda i: (0, 0)),
        ],
        out_specs=pl.BlockSpec((blk, 128), lambda i: (i, 0)),
        out_shape=jax.ShapeDtypeStruct((E, 128), F32),
    )(edge_s, w, b.reshape(1, 128))


def _node_proj(x0, w, b):
    return pl.pallas_call(
        _proj_body,
        out_shape=jax.ShapeDtypeStruct((NP, 128), F32),
    )(x0, w, b.reshape(1, 128))


def _prep_body(x_ref, wq_ref, bq_ref, wk_ref, bk_ref, wv_ref, bv_ref,
               wet_ref, a_ref, qws_ref, v_ref, *, sc):
    x = x_ref[...]
    q = jnp.dot(x, wq_ref[...], preferred_element_type=F32) + bq_ref[...]
    k = jnp.dot(x, wk_ref[...], preferred_element_type=F32) + bk_ref[...]
    v_ref[...] = jnp.dot(x, wv_ref[...], preferred_element_type=F32) + bv_ref[...]
    a_ref[...] = lax.dot_general(
        q, k, (((1,), (1,)), ((), ())), preferred_element_type=F32) * sc
    qws_ref[...] = jnp.dot(q, wet_ref[...], preferred_element_type=F32) * sc


def _prep(x, lp, fo):
    sc = 1.0 / np.sqrt(float(fo))
    return pl.pallas_call(
        functools.partial(_prep_body, sc=sc),
        out_shape=[
            jax.ShapeDtypeStruct((NP, NP), F32),
            jax.ShapeDtypeStruct((NP, 128), F32),
            jax.ShapeDtypeStruct((NP, fo), F32),
        ],
    )(x, lp['Wq'], lp['bq'].reshape(1, fo), lp['Wk'], lp['bk'].reshape(1, fo),
      lp['Wv'], lp['bv'].reshape(1, fo), lp['We'].T)


def _combine_body(s_ref, v_ref, acc_ref, den_ref, x_ref, we_ref, wsk_ref,
                  bsk_ref, o_ref, *, blk):
    i = pl.program_id(0)
    msg = jnp.dot(s_ref[...], v_ref[...], preferred_element_type=F32)
    acc = acc_ref[0] + acc_ref[1]
    msg = msg + jnp.dot(acc, we_ref[...], preferred_element_type=F32)
    den = den_ref[0] + den_ref[1]
    out = msg * (1.0 / (den + 1e-16))[:, None]
    out = out + jnp.dot(x_ref[...], wsk_ref[...], preferred_element_type=F32)
    out = out + bsk_ref[...]
    out = jnp.where(out >= 0.0, out, 0.01 * out)
    rows = i * blk + lax.broadcasted_iota(jnp.int32, (blk, 1), 0)
    o_ref[...] = jnp.where(rows < N, out, 0.0)


def _combine(S, v, acc, den, x, lp, fi, fo):
    blk = 256
    return pl.pallas_call(
        functools.partial(_combine_body, blk=blk),
        grid=(NP // blk,),
        in_specs=[
            pl.BlockSpec((blk, NP), lambda i: (i, 0)),
            pl.BlockSpec((NP, fo), lambda i: (0, 0)),
            pl.BlockSpec((2, blk, 128), lambda i: (0, i, 0)),
            pl.BlockSpec((2, blk), lambda i: (0, i)),
            pl.BlockSpec((blk, fi), lambda i: (i, 0)),
            pl.BlockSpec((128, fo), lambda i: (0, 0)),
            pl.BlockSpec((fi, fo), lambda i: (0, 0)),
            pl.BlockSpec((1, fo), lambda i: (0, 0)),
        ],
        out_specs=pl.BlockSpec((blk, fo), lambda i: (i, 0)),
        out_shape=jax.ShapeDtypeStruct((NP, fo), F32),
    )(S, v, acc, den, x, lp['We'], lp['Wskip'], lp['bskip'].reshape(1, fo))


def _head_body(x_ref, p_ref, w1_ref, b1_ref, w2_ref, b2_ref, w3_ref, b3_ref,
               o_ref):
    px = jnp.dot(p_ref[...], x_ref[...], preferred_element_type=F32)
    h = jnp.maximum(jnp.dot(px, w1_ref[...], preferred_element_type=F32)
                    + b1_ref[...], 0.0)
    h = jnp.maximum(jnp.dot(h, w2_ref[...], preferred_element_type=F32)
                    + b2_ref[...], 0.0)
    o = jnp.dot(h, w3_ref[...], preferred_element_type=F32) + b3_ref[...]
    o_ref[...] = jax.nn.sigmoid(o)


def _head(x, pool, params):
    return pl.pallas_call(
        _head_body,
        out_shape=jax.ShapeDtypeStruct((N_GRAPHS, 1), F32),
    )(x, pool, params['fc1_W'], params['fc1_b'].reshape(1, 128),
      params['fc2_W'], params['fc2_b'].reshape(1, 64),
      params['fc3_W'], params['fc3_b'].reshape(1, 1))


@functools.partial(
    pl.kernel, mesh=_mesh,
    compiler_params=pltpu.CompilerParams(needs_layout_passes=False),
    out_type=[
        jax.ShapeDtypeStruct((E,), F32),
        jax.ShapeDtypeStruct((2, NP), F32),
        jax.ShapeDtypeStruct((2, NP, 128), F32),
    ],
    scratch_types=[
        pltpu.VMEM((2, 112), jnp.int32),
        pltpu.VMEM((224,), jnp.int32),
        pltpu.VMEM((224,), jnp.int32),
        pltpu.VMEM((224,), jnp.int32),
        pltpu.VMEM((224,), F32),
        pltpu.VMEM((224, 128), F32),
        pltpu.VMEM((224, 128), F32),
        pltpu.VMEM((224,), F32),
        pltpu.VMEM((NP,), F32),
        pltpu.VMEM((16, 128), F32),
        pltpu.VMEM((128,), F32),
        pltpu.VMEM_SHARED((NP, 128), F32),
        pltpu.VMEM_SHARED((16, NP), F32),
        pltpu.SemaphoreType.DMA,
        pltpu.SemaphoreType.DMA,
    ],
)
def _edge_kernel(ea_hbm, afl_hbm, qws_hbm, src_hbm, dst_hbm,
                 exv_hbm, den_hbm, acc_hbm,
                 idx2, dstv, srcv, rowix, avr, qg, eav, exb,
                 den_l, denc, denout, accSp, denStg, sem, sem2):
    c = lax.axis_index("c")
    s = lax.axis_index("s")
    iota = lax.iota(jnp.int32, 16)
    z16 = jnp.zeros((16,), F32)
    zi16 = jnp.zeros((16,), jnp.int32)

    def zden(i, _):
        den_l[pl.ds(i * 16, 16)] = z16
        return 0
    lax.fori_loop(0, NP // 16, zden, 0)

    def zrow(i, _):
        for kk in range(8):
            eav[i, pl.ds(16 * kk, 16)] = z16
        return 0
    lax.fori_loop(0, 144, zrow, 0)
    pltpu.sync_copy(eav.at[pl.ds(0, 144)], accSp.at[pl.ds(s * 144, 144)])
    plsc.subcore_barrier()

    base = (c * 16 + s) * EPT

    def block(bb, _):
        be = base + bb * EB
        cpa = pltpu.async_copy(dst_hbm.at[pl.ds(be, 112)], idx2.at[0], sem)
        cpb = pltpu.async_copy(dst_hbm.at[pl.ds(be + 112, 112)], idx2.at[1], sem)
        cpc = pltpu.async_copy(dst_hbm.at[pl.ds(be, 224)], dstv, sem)
        cpd = pltpu.async_copy(src_hbm.at[pl.ds(be, 224)], srcv, sem)
        cpe = pltpu.async_copy(ea_hbm.at[pl.ds(be, 224)], eav, sem2)
        cpa.wait(); cpb.wait(); cpc.wait(); cpd.wait()

        def gidx(g, _):
            d16 = dstv[pl.ds(g * 16, 16)]
            s16 = srcv[pl.ds(g * 16, 16)]
            rowix[pl.ds(g * 16, 16)] = d16 * NP + s16
            return 0
        lax.fori_loop(0, 14, gidx, 0)

        cp1 = pltpu.async_copy(afl_hbm.at[rowix.at[pl.ds(0, 112)]],
                               avr.at[pl.ds(0, 112)], sem)
        cp2 = pltpu.async_copy(afl_hbm.at[rowix.at[pl.ds(112, 112)]],
                               avr.at[pl.ds(112, 112)], sem)
        cp3 = pltpu.async_copy(qws_hbm.at[idx2.at[0]], qg.at[pl.ds(0, 112)], sem)
        cp4 = pltpu.async_copy(qws_hbm.at[idx2.at[1]], qg.at[pl.ds(112, 112)], sem)
        cp1.wait(); cp2.wait(); cp3.wait(); cp4.wait(); cpe.wait()

        def group(g, _):
            e0 = g * 16
            rows = e0 + iota
            acc = avr[pl.ds(e0, 16)]
            acc2 = z16

            def dim(d8, carry):
                acc, acc2 = carry
                for u in range(0, 8, 2):
                    cols = zi16 + (d8 * 8 + u)
                    cols2 = zi16 + (d8 * 8 + u + 1)
                    a = plsc.load_gather(eav, [rows, cols])
                    b = plsc.load_gather(qg, [rows, cols])
                    a2 = plsc.load_gather(eav, [rows, cols2])
                    b2 = plsc.load_gather(qg, [rows, cols2])
                    acc = acc + a * b
                    acc2 = acc2 + a2 * b2
                return acc, acc2
            acc, acc2 = lax.fori_loop(0, 16, dim, (acc, acc2))
            exvec = jnp.exp(acc + acc2)
            exb[pl.ds(e0, 16)] = exvec
            plsc.addupdate_scatter(den_l, [dstv[pl.ds(e0, 16)]], exvec)
            return 0
        lax.fori_loop(0, 14, group, 0)

        def scale(e, _):
            exsplat = plsc.load_gather(exb, [zi16 + e])
            for kk in range(8):
                eav[e, pl.ds(16 * kk, 16)] = (
                    eav[e, pl.ds(16 * kk, 16)] * exsplat)
            return 0
        lax.fori_loop(0, EB, scale, 0)

        pltpu.sync_copy(eav.at[pl.ds(0, 112)], accSp.at[idx2.at[0]], add=True)
        pltpu.sync_copy(eav.at[pl.ds(112, 112)], accSp.at[idx2.at[1]], add=True)
        pltpu.sync_copy(exb, exv_hbm.at[pl.ds(be, 224)])
        return 0
    lax.fori_loop(0, NBLK, block, 0)

    pltpu.sync_copy(den_l, denStg.at[s])
    plsc.subcore_barrier()

    def _reduce_chunk(ch):
        pltpu.sync_copy(denStg.at[:, pl.ds(ch * 128, 128)], denc)

        def dred(m, _):
            acc = z16
            for r in range(16):
                acc = acc + denc[r, pl.ds(m * 16, 16)]
            denout[pl.ds(m * 16, 16)] = acc
            return 0
        lax.fori_loop(0, 8, dred, 0)
        pltpu.sync_copy(denout, den_hbm.at[c, pl.ds(ch * 128, 128)])

    _reduce_chunk(s)

    @pl.when(s < NP // 128 - 16)
    def _():
        _reduce_chunk(s + 16)

    pltpu.sync_copy(accSp.at[pl.ds(s * 144, 144)],
                    acc_hbm.at[c, pl.ds(s * 144, 144)])


SBS = 1792
SNB = E // SBS


@functools.partial(
    pl.kernel, mesh=_mesh,
    compiler_params=pltpu.CompilerParams(needs_layout_passes=False),
    out_type=jax.ShapeDtypeStruct((NP, NP), F32),
    scratch_types=[
        pltpu.VMEM((ROWS_PP, NP), F32),
        pltpu.VMEM((2, SBS), jnp.int32),
        pltpu.VMEM((2, SBS), jnp.int32),
        pltpu.VMEM((2, SBS), F32),
        pltpu.SemaphoreType.DMA,
        pltpu.SemaphoreType.DMA,
    ],
)
def _sbuild_kernel(src_hbm, dst_hbm, exv_hbm, zr_hbm, s_hbm,
                   sblk, srcb, dstb, exbb, semA, semB):
    c = lax.axis_index("c")
    s = lax.axis_index("s")
    wid = c * 16 + s
    sems = (semA, semB)

    def issue(b, par):
        eb = b * SBS
        pltpu.async_copy(dst_hbm.at[pl.ds(eb, SBS)], dstb.at[par], sems[par])
        pltpu.async_copy(src_hbm.at[pl.ds(eb, SBS)], srcb.at[par], sems[par])
        pltpu.async_copy(exv_hbm.at[pl.ds(eb, SBS)], exbb.at[par], sems[par])

    def drain(par):
        pltpu.make_async_copy(dst_hbm.at[pl.ds(0, SBS)], dstb.at[par],
                              sems[par]).wait()
        pltpu.make_async_copy(src_hbm.at[pl.ds(0, SBS)], srcb.at[par],
                              sems[par]).wait()
        pltpu.make_async_copy(exv_hbm.at[pl.ds(0, SBS)], exbb.at[par],
                              sems[par]).wait()

    for off, nrows in ((0, 40), (40, 32)):
        rowbase = wid * 72 + off
        pltpu.sync_copy(zr_hbm.at[pl.ds(0, nrows)], sblk.at[pl.ds(0, nrows)])
        issue(0, 0)
        issue(1, 1)

        def outer(bb, _, rowbase=rowbase, nrows=nrows):
            for par in range(2):
                b = bb * 2 + par
                drain(par)

                def grp(g, _):
                    d16 = dstb[par, pl.ds(g * 16, 16)]
                    s16 = srcb[par, pl.ds(g * 16, 16)]
                    x16 = exbb[par, pl.ds(g * 16, 16)]
                    row = d16 - rowbase
                    m = jnp.logical_and(d16 >= rowbase,
                                        d16 < rowbase + nrows)
                    plsc.addupdate_scatter(sblk, [row, s16], x16, mask=m)
                    return 0
                lax.fori_loop(0, SBS // 16, grp, 0)

                @pl.when(b + 2 < SNB)
                def _():
                    issue(b + 2, par)
            return 0
        lax.fori_loop(0, SNB // 2, outer, 0)
        pltpu.sync_copy(sblk.at[pl.ds(0, nrows)],
                        s_hbm.at[pl.ds(rowbase, nrows)])


def kernel(node_s, seq, edge_s, params, edge_index):
    src = edge_index[0]
    dst = edge_index[1]
    x0 = jnp.concatenate([node_s, seq], axis=-1).reshape(N, 26)
    x0 = jnp.pad(x0, ((0, NP - N), (0, 0)))
    x = _node_proj(x0, params['proj_node_W'], params['proj_node_b'])
    ea = _edge_proj(edge_s, params['proj_edge_W'], params['proj_edge_b'])
    zr = jnp.zeros((ROWS_PP, NP), F32)

    fis = [128, 128, 256]
    for li, (lp, fo) in enumerate(zip(params['gcn'], GCN_OUT)):
        fi = fis[li]
        A, qws, v = _prep(x, lp, fo)
        afl = A.reshape(NP * NP)
        exv, den, acc = _edge_kernel(ea, afl, qws, src, dst)
        del A
        S = _sbuild_kernel(src, dst, exv, zr)
        x = _combine(S, v, acc, den, x, lp, fi, fo)

    pool = jnp.kron(jnp.eye(N_GRAPHS, dtype=F32),
                    jnp.full((1, NODES_PER_GRAPH), 1.0 / NODES_PER_GRAPH, F32))
    return _head(x[:N], pool, params)


# speedup vs baseline: 8.2058x; 1.8520x over previous
import functools

import jax
import jax.numpy as jnp
import numpy as np
from jax import lax
from jax.experimental import pallas as pl
from jax.experimental.pallas import tpu as pltpu
from jax.experimental.pallas import tpu_sc as plsc

N_GRAPHS = 32
NODES_PER_GRAPH = 70
N = N_GRAPHS * NODES_PER_GRAPH
NP = 2304
E = 71680
GCN_OUT = [128, 256, 256]

NTILES = 32
EPT = E // NTILES
EB = 224
NBLK = EPT // EB
ROWS_PP = 40
F32 = jnp.float32

_mesh = plsc.VectorSubcoreMesh(core_axis_name="c", subcore_axis_name="s")


def _proj_body(x_ref, w_ref, b_ref, o_ref):
    o_ref[...] = (
        jnp.dot(x_ref[...], w_ref[...], preferred_element_type=F32) + b_ref[...]
    )


def _edge_proj(edge_s, w, b):
    blk = 2048
    return pl.pallas_call(
        _proj_body,
        grid=(E // blk,),
        in_specs=[
            pl.BlockSpec((blk, 39), lambda i: (i, 0)),
            pl.BlockSpec((39, 128), lambda i: (0, 0)),
            pl.BlockSpec((1, 128), lambda i: (0, 0)),
        ],
        out_specs=pl.BlockSpec((blk, 128), lambda i: (i, 0)),
        out_shape=jax.ShapeDtypeStruct((E, 128), F32),
    )(edge_s, w, b.reshape(1, 128))


def _node_proj(x0, w, b):
    return pl.pallas_call(
        _proj_body,
        out_shape=jax.ShapeDtypeStruct((NP, 128), F32),
    )(x0, w, b.reshape(1, 128))


def _prep_body(x_ref, wq_ref, bq_ref, wk_ref, bk_ref, wv_ref, bv_ref,
               wet_ref, a_ref, qws_ref, v_ref, *, sc):
    x = x_ref[...]
    q = jnp.dot(x, wq_ref[...], preferred_element_type=F32) + bq_ref[...]
    k = jnp.dot(x, wk_ref[...], preferred_element_type=F32) + bk_ref[...]
    v_ref[...] = jnp.dot(x, wv_ref[...], preferred_element_type=F32) + bv_ref[...]
    a_ref[...] = lax.dot_general(
        q, k, (((1,), (1,)), ((), ())), preferred_element_type=F32) * sc
    qws_ref[...] = jnp.dot(q, wet_ref[...], preferred_element_type=F32) * sc


def _prep(x, lp, fo):
    sc = 1.0 / np.sqrt(float(fo))
    return pl.pallas_call(
        functools.partial(_prep_body, sc=sc),
        out_shape=[
            jax.ShapeDtypeStruct((NP, NP), F32),
            jax.ShapeDtypeStruct((NP, 128), F32),
            jax.ShapeDtypeStruct((NP, fo), F32),
        ],
    )(x, lp['Wq'], lp['bq'].reshape(1, fo), lp['Wk'], lp['bk'].reshape(1, fo),
      lp['Wv'], lp['bv'].reshape(1, fo), lp['We'].T)


def _combine_body(s_ref, v_ref, acc_ref, den_ref, x_ref, we_ref, wsk_ref,
                  bsk_ref, o_ref, *, blk):
    i = pl.program_id(0)
    msg = jnp.dot(s_ref[...], v_ref[...], preferred_element_type=F32)
    acc = acc_ref[0] + acc_ref[1]
    msg = msg + jnp.dot(acc, we_ref[...], preferred_element_type=F32)
    den = den_ref[0] + den_ref[1]
    out = msg * (1.0 / (den + 1e-16))[:, None]
    out = out + jnp.dot(x_ref[...], wsk_ref[...], preferred_element_type=F32)
    out = out + bsk_ref[...]
    out = jnp.where(out >= 0.0, out, 0.01 * out)
    rows = i * blk + lax.broadcasted_iota(jnp.int32, (blk, 1), 0)
    o_ref[...] = jnp.where(rows < N, out, 0.0)


def _combine(S, v, acc, den, x, lp, fi, fo):
    blk = 256
    return pl.pallas_call(
        functools.partial(_combine_body, blk=blk),
        grid=(NP // blk,),
        in_specs=[
            pl.BlockSpec((blk, NP), lambda i: (i, 0)),
            pl.BlockSpec((NP, fo), lambda i: (0, 0)),
            pl.BlockSpec((2, blk, 128), lambda i: (0, i, 0)),
            pl.BlockSpec((2, blk), lambda i: (0, i)),
            pl.BlockSpec((blk, fi), lambda i: (i, 0)),
            pl.BlockSpec((128, fo), lambda i: (0, 0)),
            pl.BlockSpec((fi, fo), lambda i: (0, 0)),
            pl.BlockSpec((1, fo), lambda i: (0, 0)),
        ],
        out_specs=pl.BlockSpec((blk, fo), lambda i: (i, 0)),
        out_shape=jax.ShapeDtypeStruct((NP, fo), F32),
    )(S, v, acc, den, x, lp['We'], lp['Wskip'], lp['bskip'].reshape(1, fo))


def _head_body(x_ref, p_ref, w1_ref, b1_ref, w2_ref, b2_ref, w3_ref, b3_ref,
               o_ref):
    px = jnp.dot(p_ref[...], x_ref[...], preferred_element_type=F32)
    h = jnp.maximum(jnp.dot(px, w1_ref[...], preferred_element_type=F32)
                    + b1_ref[...], 0.0)
    h = jnp.maximum(jnp.dot(h, w2_ref[...], preferred_element_type=F32)
                    + b2_ref[...], 0.0)
    o = jnp.dot(h, w3_ref[...], preferred_element_type=F32) + b3_ref[...]
    o_ref[...] = jax.nn.sigmoid(o)


def _head(x, pool, params):
    return pl.pallas_call(
        _head_body,
        out_shape=jax.ShapeDtypeStruct((N_GRAPHS, 1), F32),
    )(x, pool, params['fc1_W'], params['fc1_b'].reshape(1, 128),
      params['fc2_W'], params['fc2_b'].reshape(1, 64),
      params['fc3_W'], params['fc3_b'].reshape(1, 1))


@functools.partial(
    pl.kernel, mesh=_mesh,
    compiler_params=pltpu.CompilerParams(needs_layout_passes=False),
    out_type=[
        jax.ShapeDtypeStruct((E,), F32),
        jax.ShapeDtypeStruct((2, NP), F32),
        jax.ShapeDtypeStruct((2, NP, 128), F32),
    ],
    scratch_types=[
        pltpu.VMEM((2, 112), jnp.int32),
        pltpu.VMEM((224,), jnp.int32),
        pltpu.VMEM((224,), jnp.int32),
        pltpu.VMEM((224,), jnp.int32),
        pltpu.VMEM((224,), F32),
        pltpu.VMEM((224, 128), F32),
        pltpu.VMEM((224, 128), F32),
        pltpu.VMEM((224,), F32),
        pltpu.VMEM((NP,), F32),
        pltpu.VMEM((16, 128), F32),
        pltpu.VMEM((128,), F32),
        pltpu.VMEM_SHARED((NP, 128), F32),
        pltpu.VMEM_SHARED((16, NP), F32),
        pltpu.SemaphoreType.DMA,
        pltpu.SemaphoreType.DMA,
    ],
)
def _edge_kernel(ea_hbm, afl_hbm, qws_hbm, src_hbm, dst_hbm,
                 exv_hbm, den_hbm, acc_hbm,
                 idx2, dstv, srcv, rowix, avr, qg, eav, exb,
                 den_l, denc, denout, accSp, denStg, sem, sem2):
    c = lax.axis_index("c")
    s = lax.axis_index("s")
    iota = lax.iota(jnp.int32, 16)
    z16 = jnp.zeros((16,), F32)
    zi16 = jnp.zeros((16,), jnp.int32)

    def zden(i, _):
        den_l[pl.ds(i * 16, 16)] = z16
        return 0
    lax.fori_loop(0, NP // 16, zden, 0)

    def zrow(i, _):
        for kk in range(8):
            eav[i, pl.ds(16 * kk, 16)] = z16
        return 0
    lax.fori_loop(0, 144, zrow, 0)
    pltpu.sync_copy(eav.at[pl.ds(0, 144)], accSp.at[pl.ds(s * 144, 144)])
    plsc.subcore_barrier()

    base = (c * 16 + s) * EPT

    def block(bb, _):
        be = base + bb * EB
        cpa = pltpu.async_copy(dst_hbm.at[pl.ds(be, 112)], idx2.at[0], sem)
        cpb = pltpu.async_copy(dst_hbm.at[pl.ds(be + 112, 112)], idx2.at[1], sem)
        cpc = pltpu.async_copy(dst_hbm.at[pl.ds(be, 224)], dstv, sem)
        cpd = pltpu.async_copy(src_hbm.at[pl.ds(be, 224)], srcv, sem)
        cpe = pltpu.async_copy(ea_hbm.at[pl.ds(be, 224)], eav, sem2)
        cpa.wait(); cpb.wait(); cpc.wait(); cpd.wait()

        def gidx(g, _):
            d16 = dstv[pl.ds(g * 16, 16)]
            s16 = srcv[pl.ds(g * 16, 16)]
            rowix[pl.ds(g * 16, 16)] = d16 * NP + s16
            return 0
        lax.fori_loop(0, 14, gidx, 0)

        cp1 = pltpu.async_copy(afl_hbm.at[rowix.at[pl.ds(0, 112)]],
                               avr.at[pl.ds(0, 112)], sem)
        cp2 = pltpu.async_copy(afl_hbm.at[rowix.at[pl.ds(112, 112)]],
                               avr.at[pl.ds(112, 112)], sem)
        cp3 = pltpu.async_copy(qws_hbm.at[idx2.at[0]], qg.at[pl.ds(0, 112)], sem)
        cp4 = pltpu.async_copy(qws_hbm.at[idx2.at[1]], qg.at[pl.ds(112, 112)], sem)
        cp1.wait(); cp2.wait(); cp3.wait(); cp4.wait(); cpe.wait()

        def group(g, _):
            e0 = g * 16
            av16 = avr[pl.ds(e0, 16)]
            exvec = z16
            for j in range(16):
                e = e0 + j
                r = [eav[e, pl.ds(16 * kk, 16)] for kk in range(8)]
                acc = jnp.where(iota == j, av16, 0.0)
                for kk in range(8):
                    acc = acc + r[kk] * qg[e, pl.ds(16 * kk, 16)]
                exs = jnp.exp(jnp.broadcast_to(jnp.sum(acc), (16,)))
                exvec = jnp.where(iota == j, exs, exvec)
                for kk in range(8):
                    eav[e, pl.ds(16 * kk, 16)] = r[kk] * exs
            exb[pl.ds(e0, 16)] = exvec
            plsc.addupdate_scatter(den_l, [dstv[pl.ds(e0, 16)]], exvec)
            return 0
        lax.fori_loop(0, 14, group, 0)

        pltpu.sync_copy(eav.at[pl.ds(0, 112)], accSp.at[idx2.at[0]], add=True)
        pltpu.sync_copy(eav.at[pl.ds(112, 112)], accSp.at[idx2.at[1]], add=True)
        pltpu.sync_copy(exb, exv_hbm.at[pl.ds(be, 224)])
        return 0
    lax.fori_loop(0, NBLK, block, 0)

    pltpu.sync_copy(den_l, denStg.at[s])
    plsc.subcore_barrier()

    def _reduce_chunk(ch):
        pltpu.sync_copy(denStg.at[:, pl.ds(ch * 128, 128)], denc)

        def dred(m, _):
            acc = z16
            for r in range(16):
                acc = acc + denc[r, pl.ds(m * 16, 16)]
            denout[pl.ds(m * 16, 16)] = acc
            return 0
        lax.fori_loop(0, 8, dred, 0)
        pltpu.sync_copy(denout, den_hbm.at[c, pl.ds(ch * 128, 128)])

    _reduce_chunk(s)

    @pl.when(s < NP // 128 - 16)
    def _():
        _reduce_chunk(s + 16)

    pltpu.sync_copy(accSp.at[pl.ds(s * 144, 144)],
                    acc_hbm.at[c, pl.ds(s * 144, 144)])


SBS = 1792
SNB = E // SBS


@functools.partial(
    pl.kernel, mesh=_mesh,
    compiler_params=pltpu.CompilerParams(needs_layout_passes=False),
    out_type=jax.ShapeDtypeStruct((NP, NP), F32),
    scratch_types=[
        pltpu.VMEM((ROWS_PP, NP), F32),
        pltpu.VMEM((2, SBS), jnp.int32),
        pltpu.VMEM((2, SBS), jnp.int32),
        pltpu.VMEM((2, SBS), F32),
        pltpu.SemaphoreType.DMA,
        pltpu.SemaphoreType.DMA,
    ],
)
def _sbuild_kernel(src_hbm, dst_hbm, exv_hbm, zr_hbm, s_hbm,
                   sblk, srcb, dstb, exbb, semA, semB):
    c = lax.axis_index("c")
    s = lax.axis_index("s")
    wid = c * 16 + s
    sems = (semA, semB)

    def issue(b, par):
        eb = b * SBS
        pltpu.async_copy(dst_hbm.at[pl.ds(eb, SBS)], dstb.at[par], sems[par])
        pltpu.async_copy(src_hbm.at[pl.ds(eb, SBS)], srcb.at[par], sems[par])
        pltpu.async_copy(exv_hbm.at[pl.ds(eb, SBS)], exbb.at[par], sems[par])

    def drain(par):
        pltpu.make_async_copy(dst_hbm.at[pl.ds(0, SBS)], dstb.at[par],
                              sems[par]).wait()
        pltpu.make_async_copy(src_hbm.at[pl.ds(0, SBS)], srcb.at[par],
                              sems[par]).wait()
        pltpu.make_async_copy(exv_hbm.at[pl.ds(0, SBS)], exbb.at[par],
                              sems[par]).wait()

    for off, nrows in ((0, 40), (40, 32)):
        rowbase = wid * 72 + off
        pltpu.sync_copy(zr_hbm.at[pl.ds(0, nrows)], sblk.at[pl.ds(0, nrows)])
        issue(0, 0)
        issue(1, 1)

        def outer(bb, _, rowbase=rowbase, nrows=nrows):
            for par in range(2):
                b = bb * 2 + par
                drain(par)

                def grp(g, _):
                    d16 = dstb[par, pl.ds(g * 16, 16)]
                    s16 = srcb[par, pl.ds(g * 16, 16)]
                    x16 = exbb[par, pl.ds(g * 16, 16)]
                    row = d16 - rowbase
                    m = jnp.logical_and(d16 >= rowbase,
                                        d16 < rowbase + nrows)
                    plsc.addupdate_scatter(sblk, [row, s16], x16, mask=m)
                    return 0
                lax.fori_loop(0, SBS // 16, grp, 0)

                @pl.when(b + 2 < SNB)
                def _():
                    issue(b + 2, par)
            return 0
        lax.fori_loop(0, SNB // 2, outer, 0)
        pltpu.sync_copy(sblk.at[pl.ds(0, nrows)],
                        s_hbm.at[pl.ds(rowbase, nrows)])


def kernel(node_s, seq, edge_s, params, edge_index):
    src = edge_index[0]
    dst = edge_index[1]
    x0 = jnp.concatenate([node_s, seq], axis=-1).reshape(N, 26)
    x0 = jnp.pad(x0, ((0, NP - N), (0, 0)))
    x = _node_proj(x0, params['proj_node_W'], params['proj_node_b'])
    ea = _edge_proj(edge_s, params['proj_edge_W'], params['proj_edge_b'])
    zr = jnp.zeros((ROWS_PP, NP), F32)

    fis = [128, 128, 256]
    for li, (lp, fo) in enumerate(zip(params['gcn'], GCN_OUT)):
        fi = fis[li]
        A, qws, v = _prep(x, lp, fo)
        afl = A.reshape(NP * NP)
        exv, den, acc = _edge_kernel(ea, afl, qws, src, dst)
        del A
        S = _sbuild_kernel(src, dst, exv, zr)
        x = _combine(S, v, acc, den, x, lp, fi, fo)

    pool = jnp.kron(jnp.eye(N_GRAPHS, dtype=F32),
                    jnp.full((1, NODES_PER_GRAPH), 1.0 / NODES_PER_GRAPH, F32))
    return _head(x[:N], pool, params)


# speedup vs baseline: 10.7381x; 1.3086x over previous
import functools

import jax
import jax.numpy as jnp
import numpy as np
from jax import lax
from jax.experimental import pallas as pl
from jax.experimental.pallas import tpu as pltpu
from jax.experimental.pallas import tpu_sc as plsc

N_GRAPHS = 32
NODES_PER_GRAPH = 70
N = N_GRAPHS * NODES_PER_GRAPH
NP = 2304
E = 71680
GCN_OUT = [128, 256, 256]

NTILES = 32
EPT = E // NTILES
EB = 160
HB = EB // 2
NBLK = EPT // EB
F32 = jnp.float32

_mesh = plsc.VectorSubcoreMesh(core_axis_name="c", subcore_axis_name="s")


def _proj_body(x_ref, w_ref, b_ref, o_ref):
    o_ref[...] = (
        jnp.dot(x_ref[...], w_ref[...], preferred_element_type=F32) + b_ref[...]
    )


def _edge_proj(edge_s, w, b):
    blk = 2048
    return pl.pallas_call(
        _proj_body,
        grid=(E // blk,),
        in_specs=[
            pl.BlockSpec((blk, 39), lambda i: (i, 0)),
            pl.BlockSpec((39, 128), lambda i: (0, 0)),
            pl.BlockSpec((1, 128), lambda i: (0, 0)),
        ],
        out_specs=pl.BlockSpec((blk, 128), lambda i: (i, 0)),
        out_shape=jax.ShapeDtypeStruct((E, 128), F32),
    )(edge_s, w, b.reshape(1, 128))


def _node_proj(x0, w, b):
    return pl.pallas_call(
        _proj_body,
        out_shape=jax.ShapeDtypeStruct((NP, 128), F32),
    )(x0, w, b.reshape(1, 128))


def _prep_body(x_ref, wq_ref, bq_ref, wk_ref, bk_ref, wv_ref, bv_ref,
               wet_ref, a_ref, qws_ref, v_ref, *, sc):
    x = x_ref[...]
    q = jnp.dot(x, wq_ref[...], preferred_element_type=F32) + bq_ref[...]
    k = jnp.dot(x, wk_ref[...], preferred_element_type=F32) + bk_ref[...]
    v_ref[...] = jnp.dot(x, wv_ref[...], preferred_element_type=F32) + bv_ref[...]
    a_ref[...] = lax.dot_general(
        q, k, (((1,), (1,)), ((), ())), preferred_element_type=F32) * sc
    qws_ref[...] = jnp.dot(q, wet_ref[...], preferred_element_type=F32) * sc


def _prep(x, lp, fo):
    sc = 1.0 / np.sqrt(float(fo))
    return pl.pallas_call(
        functools.partial(_prep_body, sc=sc),
        out_shape=[
            jax.ShapeDtypeStruct((NP, NP), F32),
            jax.ShapeDtypeStruct((NP, 128), F32),
            jax.ShapeDtypeStruct((NP, fo), F32),
        ],
    )(x, lp['Wq'], lp['bq'].reshape(1, fo), lp['Wk'], lp['bk'].reshape(1, fo),
      lp['Wv'], lp['bv'].reshape(1, fo), lp['We'].T)


def _combine_body(accv_ref, acc_ref, den_ref, x_ref, we_ref, wsk_ref,
                  bsk_ref, o_ref, *, blk, nv):
    i = pl.program_id(0)
    acc = acc_ref[0] + acc_ref[1]
    halves = [accv_ref[h, 0] + accv_ref[h, 1] for h in range(nv)]
    msg = halves[0] if nv == 1 else jnp.concatenate(halves, axis=1)
    msg = msg + jnp.dot(acc, we_ref[...], preferred_element_type=F32)
    den = den_ref[0] + den_ref[1]
    out = msg * (1.0 / (den + 1e-16))[:, None]
    out = out + jnp.dot(x_ref[...], wsk_ref[...], preferred_element_type=F32)
    out = out + bsk_ref[...]
    out = jnp.where(out >= 0.0, out, 0.01 * out)
    rows = i * blk + lax.broadcasted_iota(jnp.int32, (blk, 1), 0)
    o_ref[...] = jnp.where(rows < N, out, 0.0)


def _combine(accv, acc, den, x, lp, fi, fo):
    blk = 256
    nv = fo // 128
    return pl.pallas_call(
        functools.partial(_combine_body, blk=blk, nv=nv),
        grid=(NP // blk,),
        in_specs=[
            pl.BlockSpec((nv, 2, blk, 128), lambda i: (0, 0, i, 0)),
            pl.BlockSpec((2, blk, 128), lambda i: (0, i, 0)),
            pl.BlockSpec((2, blk), lambda i: (0, i)),
            pl.BlockSpec((blk, fi), lambda i: (i, 0)),
            pl.BlockSpec((128, fo), lambda i: (0, 0)),
            pl.BlockSpec((fi, fo), lambda i: (0, 0)),
            pl.BlockSpec((1, fo), lambda i: (0, 0)),
        ],
        out_specs=pl.BlockSpec((blk, fo), lambda i: (i, 0)),
        out_shape=jax.ShapeDtypeStruct((NP, fo), F32),
    )(accv, acc, den, x, lp['We'], lp['Wskip'], lp['bskip'].reshape(1, fo))


def _head_body(x_ref, p_ref, w1_ref, b1_ref, w2_ref, b2_ref, w3_ref, b3_ref,
               o_ref):
    px = jnp.dot(p_ref[...], x_ref[...], preferred_element_type=F32)
    h = jnp.maximum(jnp.dot(px, w1_ref[...], preferred_element_type=F32)
                    + b1_ref[...], 0.0)
    h = jnp.maximum(jnp.dot(h, w2_ref[...], preferred_element_type=F32)
                    + b2_ref[...], 0.0)
    o = jnp.dot(h, w3_ref[...], preferred_element_type=F32) + b3_ref[...]
    o_ref[...] = jax.nn.sigmoid(o)


def _head(x, pool, params):
    return pl.pallas_call(
        _head_body,
        out_shape=jax.ShapeDtypeStruct((N_GRAPHS, 1), F32),
    )(x, pool, params['fc1_W'], params['fc1_b'].reshape(1, 128),
      params['fc2_W'], params['fc2_b'].reshape(1, 64),
      params['fc3_W'], params['fc3_b'].reshape(1, 1))


@functools.lru_cache(maxsize=None)
def _make_edge_kernel(nv):
    @functools.partial(
        pl.kernel, mesh=_mesh,
        compiler_params=pltpu.CompilerParams(needs_layout_passes=False),
        out_type=[
            jax.ShapeDtypeStruct((2, NP), F32),
            jax.ShapeDtypeStruct((2, NP, 128), F32),
            jax.ShapeDtypeStruct((nv, 2, NP, 128), F32),
        ],
        scratch_types=[
            pltpu.VMEM((EPT,), jnp.int32),
            pltpu.VMEM((EPT,), jnp.int32),
            pltpu.VMEM((EPT,), F32),
            pltpu.VMEM((EB,), jnp.int32),
            pltpu.VMEM((EB,), F32),
            pltpu.VMEM((EB, 128), F32),
            pltpu.VMEM((EB, 128), F32),
            pltpu.VMEM((EB, 128), F32),
            pltpu.VMEM((NP,), F32),
            pltpu.VMEM((16, 128), F32),
            pltpu.VMEM((128,), F32),
            pltpu.VMEM_SHARED((NP, 128), F32),
            pltpu.VMEM_SHARED((NP, 128), F32),
            pltpu.VMEM_SHARED((16, NP), F32),
            pltpu.SemaphoreType.DMA,
            pltpu.SemaphoreType.DMA,
            pltpu.SemaphoreType.DMA,
        ],
    )
    def _edge_kernel(ea_hbm, afl_hbm, qws_hbm, v1_hbm, v2_hbm, src_hbm,
                     dst_hbm, den_hbm, acc_hbm, accv_hbm,
                     dstA, srcA, exA, rowix, avr, qg, eav, vv,
                     den_l, denc, denout, accSp, accVp, denStg,
                     sem, sem2, sem3):
        c = lax.axis_index("c")
        s = lax.axis_index("s")
        iota = lax.iota(jnp.int32, 16)
        z16 = jnp.zeros((16,), F32)
        base = (c * 16 + s) * EPT

        cpi = pltpu.async_copy(dst_hbm.at[pl.ds(base, EPT)], dstA, sem)
        cpj = pltpu.async_copy(src_hbm.at[pl.ds(base, EPT)], srcA, sem)

        def zden(i, _):
            den_l[pl.ds(i * 16, 16)] = z16
            return 0
        lax.fori_loop(0, NP // 16, zden, 0)

        def zrow(i, _):
            for kk in range(8):
                eav[i, pl.ds(16 * kk, 16)] = z16
                vv[i, pl.ds(16 * kk, 16)] = z16
            return 0
        lax.fori_loop(0, 144, zrow, 0)
        pltpu.sync_copy(eav.at[pl.ds(0, 144)], accSp.at[pl.ds(s * 144, 144)])
        pltpu.sync_copy(vv.at[pl.ds(0, 144)], accVp.at[pl.ds(s * 144, 144)])
        cpi.wait(); cpj.wait()
        plsc.subcore_barrier()

        def block(bb, _):
            bo = bb * EB
            cpe = pltpu.async_copy(ea_hbm.at[pl.ds(base + bo, EB)], eav, sem2)

            def gidx(g, _):
                d16 = dstA[pl.ds(bo + g * 16, 16)]
                s16 = srcA[pl.ds(bo + g * 16, 16)]
                rowix[pl.ds(g * 16, 16)] = d16 * NP + s16
                return 0
            lax.fori_loop(0, EB // 16, gidx, 0)

            cp1 = pltpu.async_copy(afl_hbm.at[rowix.at[pl.ds(0, HB)]],
                                   avr.at[pl.ds(0, HB)], sem)
            cp2 = pltpu.async_copy(afl_hbm.at[rowix.at[pl.ds(HB, HB)]],
                                   avr.at[pl.ds(HB, HB)], sem)
            cp3 = pltpu.async_copy(qws_hbm.at[dstA.at[pl.ds(bo, HB)]],
                                   qg.at[pl.ds(0, HB)], sem)
            cp4 = pltpu.async_copy(qws_hbm.at[dstA.at[pl.ds(bo + HB, HB)]],
                                   qg.at[pl.ds(HB, HB)], sem)
            cp5 = pltpu.async_copy(v1_hbm.at[srcA.at[pl.ds(bo, HB)]],
                                   vv.at[pl.ds(0, HB)], sem3)
            cp6 = pltpu.async_copy(v1_hbm.at[srcA.at[pl.ds(bo + HB, HB)]],
                                   vv.at[pl.ds(HB, HB)], sem3)
            cp1.wait(); cp2.wait(); cp3.wait(); cp4.wait(); cpe.wait()
            cp5.wait(); cp6.wait()

            def group(g, _):
                e0 = g * 16
                av16 = avr[pl.ds(e0, 16)]
                exvec = z16
                for j in range(16):
                    e = e0 + j
                    r = [eav[e, pl.ds(16 * kk, 16)] for kk in range(8)]
                    acc = jnp.where(iota == j, av16, 0.0)
                    for kk in range(8):
                        acc = acc + r[kk] * qg[e, pl.ds(16 * kk, 16)]
                    exs = jnp.exp(jnp.broadcast_to(jnp.sum(acc), (16,)))
                    exvec = jnp.where(iota == j, exs, exvec)
                    for kk in range(8):
                        eav[e, pl.ds(16 * kk, 16)] = r[kk] * exs
                        vv[e, pl.ds(16 * kk, 16)] = vv[e, pl.ds(16 * kk, 16)] * exs
                exA[pl.ds(bo + e0, 16)] = exvec
                plsc.addupdate_scatter(den_l, [dstA[pl.ds(bo + e0, 16)]], exvec)
                return 0
            lax.fori_loop(0, EB // 16, group, 0)

            pltpu.sync_copy(eav.at[pl.ds(0, HB)],
                            accSp.at[dstA.at[pl.ds(bo, HB)]], add=True)
            pltpu.sync_copy(eav.at[pl.ds(HB, HB)],
                            accSp.at[dstA.at[pl.ds(bo + HB, HB)]], add=True)
            pltpu.sync_copy(vv.at[pl.ds(0, HB)],
                            accVp.at[dstA.at[pl.ds(bo, HB)]], add=True)
            pltpu.sync_copy(vv.at[pl.ds(HB, HB)],
                            accVp.at[dstA.at[pl.ds(bo + HB, HB)]], add=True)
            return 0
        lax.fori_loop(0, NBLK, block, 0)

        pltpu.sync_copy(den_l, denStg.at[s])
        plsc.subcore_barrier()

        def _reduce_chunk(ch):
            pltpu.sync_copy(denStg.at[:, pl.ds(ch * 128, 128)], denc)

            def dred(m, _):
                acc = z16
                for r in range(16):
                    acc = acc + denc[r, pl.ds(m * 16, 16)]
                denout[pl.ds(m * 16, 16)] = acc
                return 0
            lax.fori_loop(0, 8, dred, 0)
            pltpu.sync_copy(denout, den_hbm.at[c, pl.ds(ch * 128, 128)])

        _reduce_chunk(s)

        @pl.when(s < NP // 128 - 16)
        def _():
            _reduce_chunk(s + 16)

        pltpu.sync_copy(accSp.at[pl.ds(s * 144, 144)],
                        acc_hbm.at[c, pl.ds(s * 144, 144)])
        pltpu.sync_copy(accVp.at[pl.ds(s * 144, 144)],
                        accv_hbm.at[0, c, pl.ds(s * 144, 144)])

        if nv == 2:
            def zrow2(i, _):
                for kk in range(8):
                    vv[i, pl.ds(16 * kk, 16)] = z16
                return 0
            lax.fori_loop(0, 144, zrow2, 0)
            pltpu.sync_copy(vv.at[pl.ds(0, 144)], accVp.at[pl.ds(s * 144, 144)])
            plsc.subcore_barrier()

            def blockB(bb, _):
                bo = bb * EB
                cp5 = pltpu.async_copy(v2_hbm.at[srcA.at[pl.ds(bo, HB)]],
                                       vv.at[pl.ds(0, HB)], sem3)
                cp6 = pltpu.async_copy(v2_hbm.at[srcA.at[pl.ds(bo + HB, HB)]],
                                       vv.at[pl.ds(HB, HB)], sem3)
                cp5.wait(); cp6.wait()

                def groupB(g, _):
                    e0 = g * 16
                    ex16 = exA[pl.ds(bo + e0, 16)]
                    for j in range(16):
                        e = e0 + j
                        exs = jnp.broadcast_to(
                            jnp.sum(jnp.where(iota == j, ex16, 0.0)), (16,))
                        for kk in range(8):
                            vv[e, pl.ds(16 * kk, 16)] = (
                                vv[e, pl.ds(16 * kk, 16)] * exs)
                    return 0
                lax.fori_loop(0, EB // 16, groupB, 0)

                pltpu.sync_copy(vv.at[pl.ds(0, HB)],
                                accVp.at[dstA.at[pl.ds(bo, HB)]], add=True)
                pltpu.sync_copy(vv.at[pl.ds(HB, HB)],
                                accVp.at[dstA.at[pl.ds(bo + HB, HB)]], add=True)
                return 0
            lax.fori_loop(0, NBLK, blockB, 0)

            plsc.subcore_barrier()
            pltpu.sync_copy(accVp.at[pl.ds(s * 144, 144)],
                            accv_hbm.at[1, c, pl.ds(s * 144, 144)])

    return _edge_kernel


def kernel(node_s, seq, edge_s, params, edge_index):
    src = edge_index[0]
    dst = edge_index[1]
    x0 = jnp.concatenate([node_s, seq], axis=-1).reshape(N, 26)
    x0 = jnp.pad(x0, ((0, NP - N), (0, 0)))
    x = _node_proj(x0, params['proj_node_W'], params['proj_node_b'])
    ea = _edge_proj(edge_s, params['proj_edge_W'], params['proj_edge_b'])

    fis = [128, 128, 256]
    for li, (lp, fo) in enumerate(zip(params['gcn'], GCN_OUT)):
        fi = fis[li]
        nv = fo // 128
        A, qws, v = _prep(x, lp, fo)
        afl = A.reshape(NP * NP)
        v1 = v[:, :128]
        v2 = v[:, 128:] if nv == 2 else v1
        den, acc, accv = _make_edge_kernel(nv)(ea, afl, qws, v1, v2, src, dst)
        del A
        x = _combine(accv, acc, den, x, lp, fi, fo)

    pool = jnp.kron(jnp.eye(N_GRAPHS, dtype=F32),
                    jnp.full((1, NODES_PER_GRAPH), 1.0 / NODES_PER_GRAPH, F32))
    return _head(x[:N], pool, params)
